# baseline (speedup 1.0000x reference)
import jax
import jax.numpy as jnp
import numpy as np
from jax import lax
from jax.experimental import pallas as pl
from jax.experimental.pallas import tpu as pltpu
from jax.experimental.pallas import tpu_sc as plsc

N_POINTS = 262144
N_LEVELS = 16
F_PER_LEVEL = 2
LOG2_T = 19
T = 1 << LOG2_T
MASK = T - 1
BASE_RES = 16
PER_LEVEL_SCALE = 1.3819129

P1 = int(np.uint32(2654435761).view(np.int32))
P2 = int(np.uint32(805459861).view(np.int32))

RES = [float(np.floor(BASE_RES * (PER_LEVEL_SCALE ** l))) for l in range(N_LEVELS)]

NC = 2
NS = 16
LANES = 16
NW = NC * NS
PPW = N_POINTS // NW
B = 512
G = B // LANES
NCH = PPW // B
NF = N_LEVELS * F_PER_LEVEL

CORNERS = [(i, j, k) for i in (0, 1) for j in (0, 1) for k in (0, 1)]


def _body(x0_hbm, x1_hbm, x2_hbm, tab0_hbm, tab1_hbm, out_hbm,
          x_v, frac_v, idx_v, rows0_v, rows1_v, out_v, sem0, sem1):
    wid = lax.axis_index("s") * NC + lax.axis_index("c")
    lane = lax.iota(jnp.int32, 16)
    zeros16 = lane * 0
    sems = (sem0, sem1)

    def fill_idx(g, _):
        o = g * LANES
        v = ((lane + o + wid * 977) * 2053 & (T * 8 - 1)) * 2
        pos = (lane + o) * 2
        plsc.store_scatter(idx_v.at[0], [pos], v)
        plsc.store_scatter(idx_v.at[0], [pos + 1], v + 1)
        plsc.store_scatter(idx_v.at[1], [pos], v)
        plsc.store_scatter(idx_v.at[1], [pos + 1], v + 1)
        return 0

    lax.fori_loop(0, 8 * B // LANES, fill_idx, 0)

    def chunk_body(c, carry):
        base = wid * PPW + c * B
        pltpu.sync_copy(x0_hbm.at[pl.ds(base, B)], x_v.at[0])
        pltpu.sync_copy(x1_hbm.at[pl.ds(base, B)], x_v.at[1])
        pltpu.sync_copy(x2_hbm.at[pl.ds(base, B)], x_v.at[2])

        def hash_level(l, s):
            res = RES[l]

            def hash_body(g, _):
                o = g * LANES
                x0 = x_v[0, pl.ds(o, LANES)] * res
                x1 = x_v[1, pl.ds(o, LANES)] * res
                x2 = x_v[2, pl.ds(o, LANES)] * res
                p0 = x0.astype(jnp.int32)
                p1 = x1.astype(jnp.int32)
                p2 = x2.astype(jnp.int32)
                frac_v[s, 0, pl.ds(o, LANES)] = x0 - p0.astype(jnp.float32)
                frac_v[s, 1, pl.ds(o, LANES)] = x1 - p1.astype(jnp.float32)
                frac_v[s, 2, pl.ds(o, LANES)] = x2 - p2.astype(jnp.float32)
                hx = (p0, p0 + 1)
                hy0 = p1 * P1
                hy = (hy0, hy0 + P1)
                hz0 = p2 * P2
                hz = (hz0, hz0 + P2)
                for ci, (i, j, k) in enumerate(CORNERS):
                    h = (hx[i] ^ hy[j] ^ hz[k]) & MASK
                    idx_v[s, pl.ds(ci * B + o, LANES)] = h + l * T
                return 0

            lax.fori_loop(0, G, hash_body, 0)

        def fire(s):
            d0 = pltpu.async_copy(tab0_hbm.at[idx_v.at[s]], rows0_v.at[s], sems[s])
            return d0, d0

        def combine(l, s):
            def comb_body(g, _):
                o = g * LANES
                fx = frac_v[s, 0, pl.ds(o, LANES)]
                fy = frac_v[s, 1, pl.ds(o, LANES)]
                fz = frac_v[s, 2, pl.ds(o, LANES)]
                wx = (1.0 - fx, fx)
                wy = (1.0 - fy, fy)
                wz = (1.0 - fz, fz)
                acc0 = jnp.zeros((16,), jnp.float32)
                acc1 = jnp.zeros((16,), jnp.float32)
                for ci, (i, j, k) in enumerate(CORNERS):
                    w = wx[i] * wy[j] * wz[k]
                    f0 = rows0_v[s, pl.ds(ci * B + o, LANES)]
                    f1 = rows1_v[s, pl.ds(ci * B + o, LANES)]
                    acc0 = acc0 + w * f0
                    acc1 = acc1 + w * f1
                nidx = lane + o
                plsc.store_scatter(out_v, [nidx, zeros16 + (2 * l)], acc0)
                plsc.store_scatter(out_v, [nidx, zeros16 + (2 * l + 1)], acc1)
                return 0

            lax.fori_loop(0, G, comb_body, 0)

        descs = {0: fire(0)}
        for l in range(N_LEVELS):
            s = l % 2
            if l + 1 < N_LEVELS:
                ns = (l + 1) % 2
                descs[ns] = fire(ns)
            d0, _ = descs[s]
            d0.wait()

        pltpu.sync_copy(out_v, out_hbm.at[pl.ds(base, B)])
        return carry

    lax.fori_loop(0, NCH, chunk_body, 0)


@jax.jit
def _encode_sc(x0, x1, x2, tab0, tab1):
    mesh = plsc.VectorSubcoreMesh(core_axis_name="c", subcore_axis_name="s")
    return pl.kernel(
        _body,
        out_type=jax.ShapeDtypeStruct((N_POINTS, NF), jnp.float32),
        mesh=mesh,
        compiler_params=pltpu.CompilerParams(
            needs_layout_passes=False, use_tc_tiling_on_sc=False
        ),
        scratch_types=[
            pltpu.VMEM((3, B), jnp.float32),
            pltpu.VMEM((2, 3, B), jnp.float32),
            pltpu.VMEM((2, 16 * B), jnp.int32),
            pltpu.VMEM((2, 16 * B), jnp.float32),
            pltpu.VMEM((2, 8 * B), jnp.float32),
            pltpu.VMEM((B, NF), jnp.float32),
            pltpu.SemaphoreType.DMA,
            pltpu.SemaphoreType.DMA,
        ],
    )(x0, x1, x2, tab0, tab1)


def kernel(x, table):
    x = x.astype(jnp.float32)
    tabI = table.reshape(N_LEVELS * T * F_PER_LEVEL)
    return _encode_sc(x[:, 0], x[:, 1], x[:, 2], tabI, tabI)


# speedup vs baseline: 4.9249x; 4.9249x over previous
import jax
import jax.numpy as jnp
import numpy as np
from jax import lax
from jax.experimental import pallas as pl
from jax.experimental.pallas import tpu as pltpu
from jax.experimental.pallas import tpu_sc as plsc

N_POINTS = 262144
N_LEVELS = 16
F_PER_LEVEL = 2
LOG2_T = 19
T = 1 << LOG2_T
MASK = T - 1
BASE_RES = 16
PER_LEVEL_SCALE = 1.3819129

P1 = int(np.uint32(2654435761).view(np.int32))
P2 = int(np.uint32(805459861).view(np.int32))

RES = [float(np.floor(BASE_RES * (PER_LEVEL_SCALE ** l))) for l in range(N_LEVELS)]

NC = 2
NS = 16
LANES = 16
NW = NC * NS
PPW = N_POINTS // NW
B = 512
G = B // LANES
NCH = PPW // B
NF = N_LEVELS * F_PER_LEVEL

CORNERS = [(i, j, k) for i in (0, 1) for j in (0, 1) for k in (0, 1)]


def _body(x0_hbm, x1_hbm, x2_hbm, tab0_hbm, tab1_hbm, out_hbm,
          x_v, frac_v, idx_v, rows0_v, rows1_v, out_v, sem0, sem1):
    wid = lax.axis_index("s") * NC + lax.axis_index("c")
    lane = lax.iota(jnp.int32, 16)
    zeros16 = lane * 0
    sems = (sem0, sem1)

    def fill_idx(g, _):
        o = g * LANES
        v = ((lane + o + wid * 977) * 134775813) & (N_LEVELS * T - 1)
        idx_v[0, pl.ds(o, LANES)] = v
        idx_v[1, pl.ds(o, LANES)] = v
        return 0

    lax.fori_loop(0, 8 * B // LANES, fill_idx, 0)

    def chunk_body(c, carry):
        base = wid * PPW + c * B
        pltpu.sync_copy(x0_hbm.at[pl.ds(base, B)], x_v.at[0])
        pltpu.sync_copy(x1_hbm.at[pl.ds(base, B)], x_v.at[1])
        pltpu.sync_copy(x2_hbm.at[pl.ds(base, B)], x_v.at[2])

        def hash_level(l, s):
            res = RES[l]

            def hash_body(g, _):
                o = g * LANES
                x0 = x_v[0, pl.ds(o, LANES)] * res
                x1 = x_v[1, pl.ds(o, LANES)] * res
                x2 = x_v[2, pl.ds(o, LANES)] * res
                p0 = x0.astype(jnp.int32)
                p1 = x1.astype(jnp.int32)
                p2 = x2.astype(jnp.int32)
                frac_v[s, 0, pl.ds(o, LANES)] = x0 - p0.astype(jnp.float32)
                frac_v[s, 1, pl.ds(o, LANES)] = x1 - p1.astype(jnp.float32)
                frac_v[s, 2, pl.ds(o, LANES)] = x2 - p2.astype(jnp.float32)
                hx = (p0, p0 + 1)
                hy0 = p1 * P1
                hy = (hy0, hy0 + P1)
                hz0 = p2 * P2
                hz = (hz0, hz0 + P2)
                for ci, (i, j, k) in enumerate(CORNERS):
                    h = (hx[i] ^ hy[j] ^ hz[k]) & MASK
                    idx_v[s, pl.ds(ci * B + o, LANES)] = h + l * T
                return 0

            lax.fori_loop(0, G, hash_body, 0)

        def fire(s):
            d0 = pltpu.async_copy(tab0_hbm.at[idx_v.at[s]], rows0_v.at[s], sems[s])
            d1 = pltpu.async_copy(tab1_hbm.at[idx_v.at[s]], rows1_v.at[s], sems[s])
            return d0, d1

        def combine(l, s):
            def comb_body(g, _):
                o = g * LANES
                fx = frac_v[s, 0, pl.ds(o, LANES)]
                fy = frac_v[s, 1, pl.ds(o, LANES)]
                fz = frac_v[s, 2, pl.ds(o, LANES)]
                wx = (1.0 - fx, fx)
                wy = (1.0 - fy, fy)
                wz = (1.0 - fz, fz)
                acc0 = jnp.zeros((16,), jnp.float32)
                acc1 = jnp.zeros((16,), jnp.float32)
                for ci, (i, j, k) in enumerate(CORNERS):
                    w = wx[i] * wy[j] * wz[k]
                    f0 = rows0_v[s, pl.ds(ci * B + o, LANES)]
                    f1 = rows1_v[s, pl.ds(ci * B + o, LANES)]
                    acc0 = acc0 + w * f0
                    acc1 = acc1 + w * f1
                nidx = lane + o
                plsc.store_scatter(out_v, [nidx, zeros16 + (2 * l)], acc0)
                plsc.store_scatter(out_v, [nidx, zeros16 + (2 * l + 1)], acc1)
                return 0

            lax.fori_loop(0, G, comb_body, 0)

        descs = {0: fire(0)}
        for l in range(N_LEVELS):
            s = l % 2
            if l + 1 < N_LEVELS:
                ns = (l + 1) % 2
                descs[ns] = fire(ns)
            d0, d1 = descs[s]
            d0.wait()
            d1.wait()

        pltpu.sync_copy(out_v, out_hbm.at[pl.ds(base, B)])
        return carry

    lax.fori_loop(0, NCH, chunk_body, 0)


@jax.jit
def _encode_sc(x0, x1, x2, tab0, tab1):
    mesh = plsc.VectorSubcoreMesh(core_axis_name="c", subcore_axis_name="s")
    return pl.kernel(
        _body,
        out_type=jax.ShapeDtypeStruct((N_POINTS, NF), jnp.float32),
        mesh=mesh,
        compiler_params=pltpu.CompilerParams(
            needs_layout_passes=False, use_tc_tiling_on_sc=False
        ),
        scratch_types=[
            pltpu.VMEM((3, B), jnp.float32),
            pltpu.VMEM((2, 3, B), jnp.float32),
            pltpu.VMEM((2, 8 * B), jnp.int32),
            pltpu.VMEM((2, 8 * B), jnp.float32),
            pltpu.VMEM((2, 8 * B), jnp.float32),
            pltpu.VMEM((B, NF), jnp.float32),
            pltpu.SemaphoreType.DMA,
            pltpu.SemaphoreType.DMA,
        ],
    )(x0, x1, x2, tab0, tab1)


def kernel(x, table):
    x = x.astype(jnp.float32)
    tab0 = table[:, :, 0].reshape(N_LEVELS * T)
    tab1 = table[:, :, 1].reshape(N_LEVELS * T)
    return _encode_sc(x[:, 0], x[:, 1], x[:, 2], tab0, tab1)


# speedup vs baseline: 8.5077x; 1.7275x over previous
import jax
import jax.numpy as jnp
import numpy as np
from jax import lax
from jax.experimental import pallas as pl
from jax.experimental.pallas import tpu as pltpu
from jax.experimental.pallas import tpu_sc as plsc

N_POINTS = 262144
N_LEVELS = 16
F_PER_LEVEL = 2
LOG2_T = 19
T = 1 << LOG2_T
MASK = T - 1
BASE_RES = 16
PER_LEVEL_SCALE = 1.3819129

P1 = int(np.uint32(2654435761).view(np.int32))
P2 = int(np.uint32(805459861).view(np.int32))

RES = [float(np.floor(BASE_RES * (PER_LEVEL_SCALE ** l))) for l in range(N_LEVELS)]

NC = 2
NS = 16
LANES = 16
NW = NC * NS
PPW = N_POINTS // NW
B = 512
G = B // LANES
NCH = PPW // B
NF = N_LEVELS * F_PER_LEVEL

CORNERS = [(i, j, k) for i in (0, 1) for j in (0, 1) for k in (0, 1)]


def _body(x0_hbm, x1_hbm, x2_hbm, tab_hbm, out_hbm,
          x_v, frac_v, idx_v, rows_v, out_v, sem0, sem1):
    wid = lax.axis_index("s") * NC + lax.axis_index("c")
    lane = lax.iota(jnp.int32, 16)
    zeros16 = lane * 0
    sems = (sem0, sem1)

    def chunk_body(c, carry):
        base = wid * PPW + c * B
        pltpu.sync_copy(x0_hbm.at[pl.ds(base, B)], x_v.at[0])
        pltpu.sync_copy(x1_hbm.at[pl.ds(base, B)], x_v.at[1])
        pltpu.sync_copy(x2_hbm.at[pl.ds(base, B)], x_v.at[2])

        def hash_level(l, s):
            res = RES[l]

            def hash_body(g, _):
                o = g * LANES
                x0 = x_v[0, pl.ds(o, LANES)] * res
                x1 = x_v[1, pl.ds(o, LANES)] * res
                x2 = x_v[2, pl.ds(o, LANES)] * res
                p0 = x0.astype(jnp.int32)
                p1 = x1.astype(jnp.int32)
                p2 = x2.astype(jnp.int32)
                frac_v[s, 0, pl.ds(o, LANES)] = x0 - p0.astype(jnp.float32)
                frac_v[s, 1, pl.ds(o, LANES)] = x1 - p1.astype(jnp.float32)
                frac_v[s, 2, pl.ds(o, LANES)] = x2 - p2.astype(jnp.float32)
                hx = (p0, p0 + 1)
                hy0 = p1 * P1
                hy = (hy0, hy0 + P1)
                hz0 = p2 * P2
                hz = (hz0, hz0 + P2)
                for ci, (i, j, k) in enumerate(CORNERS):
                    h = (hx[i] ^ hy[j] ^ hz[k]) & MASK
                    idx_v[s, pl.ds(ci * B + o, LANES)] = h + l * T
                return 0

            lax.fori_loop(0, G, hash_body, 0)

        def fire(s):
            return pltpu.async_copy(tab_hbm.at[idx_v.at[s]], rows_v.at[s], sems[s])

        def combine(l, s):
            def comb_body(g, _):
                o = g * LANES
                fx = frac_v[s, 0, pl.ds(o, LANES)]
                fy = frac_v[s, 1, pl.ds(o, LANES)]
                fz = frac_v[s, 2, pl.ds(o, LANES)]
                wx = (1.0 - fx, fx)
                wy = (1.0 - fy, fy)
                wz = (1.0 - fz, fz)
                acc0 = jnp.zeros((16,), jnp.float32)
                acc1 = jnp.zeros((16,), jnp.float32)
                for ci, (i, j, k) in enumerate(CORNERS):
                    w = wx[i] * wy[j] * wz[k]
                    v = rows_v[s, pl.ds(ci * B + o, LANES)]
                    f0 = lax.bitcast_convert_type(v & jnp.int32(-65536), jnp.float32)
                    f1 = lax.bitcast_convert_type(v << 16, jnp.float32)
                    acc0 = acc0 + w * f0
                    acc1 = acc1 + w * f1
                nidx = lane + o
                plsc.store_scatter(out_v, [nidx, zeros16 + (2 * l)], acc0)
                plsc.store_scatter(out_v, [nidx, zeros16 + (2 * l + 1)], acc1)
                return 0

            lax.fori_loop(0, G, comb_body, 0)

        hash_level(0, 0)
        descs = {0: fire(0)}
        for l in range(N_LEVELS):
            s = l % 2
            if l + 1 < N_LEVELS:
                ns = (l + 1) % 2
                hash_level(l + 1, ns)
                descs[ns] = fire(ns)
            descs[s].wait()
            combine(l, s)

        pltpu.sync_copy(out_v, out_hbm.at[pl.ds(base, B)])
        return carry

    lax.fori_loop(0, NCH, chunk_body, 0)


@jax.jit
def _encode_sc(x0, x1, x2, tab):
    mesh = plsc.VectorSubcoreMesh(core_axis_name="c", subcore_axis_name="s")
    return pl.kernel(
        _body,
        out_type=jax.ShapeDtypeStruct((N_POINTS, NF), jnp.float32),
        mesh=mesh,
        compiler_params=pltpu.CompilerParams(
            needs_layout_passes=False, use_tc_tiling_on_sc=False
        ),
        scratch_types=[
            pltpu.VMEM((3, B), jnp.float32),
            pltpu.VMEM((2, 3, B), jnp.float32),
            pltpu.VMEM((2, 8 * B), jnp.int32),
            pltpu.VMEM((2, 8 * B), jnp.int32),
            pltpu.VMEM((B, NF), jnp.float32),
            pltpu.SemaphoreType.DMA,
            pltpu.SemaphoreType.DMA,
        ],
    )(x0, x1, x2, tab)


def kernel(x, table):
    x = x.astype(jnp.float32)
    b0 = lax.bitcast_convert_type(
        table[:, :, 0].astype(jnp.bfloat16), jnp.uint16).astype(jnp.int32)
    b1 = lax.bitcast_convert_type(
        table[:, :, 1].astype(jnp.bfloat16), jnp.uint16).astype(jnp.int32)
    tab = ((b0 << 16) | b1).reshape(N_LEVELS * T)
    return _encode_sc(x[:, 0], x[:, 1], x[:, 2], tab)
